# baseline (speedup 1.0000x reference)
import functools

import jax
import jax.numpy as jnp
from jax import lax
from jax.experimental import pallas as pl
from jax.experimental.pallas import tpu as pltpu
from jax.experimental.pallas import tpu_sc as plsc

L = 128
H = 512
BSZ = 16
N = 20
NPAD = 32
TOPK = 5


def _scores_kernel(q_ref, ka_ref, kb_ref, out_ref):
    g = pl.program_id(0)

    @pl.when(g == 0)
    def _():
        out_ref[...] = jnp.full((BSZ, NPAD), -jnp.inf, dtype=jnp.float32)

    batch = jax.lax.broadcasted_iota(jnp.int32, (BSZ, 1), 0)
    slot = jax.lax.broadcasted_iota(jnp.int32, (BSZ, NPAD), 1)
    for half, k_ref in ((0, ka_ref), (1, kb_ref)):
        k3 = k_ref[0].reshape(BSZ, L, H)
        acc = jnp.full((BSZ, 1), -jnp.inf, dtype=jnp.float32)
        for b in range(BSZ):
            att = jax.lax.dot_general(
                k3[b], q_ref[:, b, :], (((1,), (1,)), ((), ())),
                preferred_element_type=jnp.float32)
            acc = jnp.where(batch == b, jnp.max(att), acc)
        out_ref[...] = jnp.where(slot == 2 * g + half, acc, out_ref[...])


_SC_INFO = plsc.get_sparse_core_info()
_NC = _SC_INFO.num_cores


_GATHER_DNUMS = lax.GatherDimensionNumbers(
    offset_dims=(), collapsed_slice_dims=(0,), start_index_map=(0,))


def _shuffle(v, col0, sh):
    return lax.gather(v, (col0 ^ sh).reshape(16, 1), _GATHER_DNUMS, (1,),
                      mode=lax.GatherScatterMode.PROMISE_IN_BOUNDS)


def _allmax(v, col0):
    for sh in (1, 2, 4, 8):
        v = jnp.maximum(v, _shuffle(v, col0, sh))
    return v


def _allmin(v, col0):
    for sh in (1, 2, 4, 8):
        v = jnp.minimum(v, _shuffle(v, col0, sh))
    return v


def _topk_sc_kernel(s_hbm, out_hbm, row_v, outrow_v, sem):
    wid = lax.axis_index("s") * _NC + lax.axis_index("c")

    @pl.when(wid < BSZ)
    def _():
        pltpu.sync_copy(s_hbm.at[wid], row_v)
        col0 = lax.iota(jnp.int32, 16)
        col1 = col0 + 16
        row0 = row_v[pl.ds(0, 16)]
        row1 = row_v[pl.ds(16, 16)]
        outv = jnp.zeros((16,), jnp.int32)
        for k in range(TOPK):
            m = jnp.maximum(_allmax(row0, col0), _allmax(row1, col0))
            i0 = _allmin(jnp.where(row0 == m, col0, NPAD), col0)
            i1 = _allmin(jnp.where(row1 == m, col1, NPAD), col0)
            idx = jnp.minimum(i0, i1)
            outv = jnp.where(col0 == k, idx, outv)
            row0 = jnp.where(col0 == idx, -jnp.inf, row0)
            row1 = jnp.where(col1 == idx, -jnp.inf, row1)
        outrow_v[...] = outv
        pltpu.sync_copy(outrow_v, out_hbm.at[wid])


def kernel(query, keys, values):
    del values
    q3 = query.reshape(L, BSZ, H)

    scores = pl.pallas_call(
        _scores_kernel,
        grid=(N // 2,),
        in_specs=[
            pl.BlockSpec((L, BSZ, H), lambda g: (0, 0, 0)),
            pl.BlockSpec((1, BSZ, L * H), lambda g: (2 * g, 0, 0)),
            pl.BlockSpec((1, BSZ, L * H), lambda g: (2 * g + 1, 0, 0)),
        ],
        out_specs=pl.BlockSpec((BSZ, NPAD), lambda g: (0, 0)),
        out_shape=jax.ShapeDtypeStruct((BSZ, NPAD), jnp.float32),
    )(q3, keys, keys)

    topk_rows = pl.kernel(
        _topk_sc_kernel,
        out_type=jax.ShapeDtypeStruct((BSZ, 16), jnp.int32),
        mesh=plsc.VectorSubcoreMesh(core_axis_name="c", subcore_axis_name="s"),
        scratch_types=[
            pltpu.VMEM((NPAD,), jnp.float32),
            pltpu.VMEM((16,), jnp.int32),
            pltpu.SemaphoreType.DMA,
        ],
    )(scores)

    attention = scores[:, :N].reshape(BSZ, 1, N)
    topk_idx = topk_rows[:, :TOPK].T
    return (attention, topk_idx)
